# baseline (speedup 1.0000x reference)
import functools

import jax
import jax.numpy as jnp
from jax import lax
from jax.experimental import pallas as pl
from jax.experimental.pallas import tpu as pltpu
from jax.experimental.pallas import tpu_sc as plsc

D_EMB = 64
NC = 2
NS = 16
NW = NC * NS
CHUNK = 128
NBUF = 2


def _embed_body(idx_hbm, table_hbm, out_hbm, idx_v, buf0, buf1, sem0, sem1):
    n_chunks = idx_hbm.shape[1]
    wid = lax.axis_index("s") * NC + lax.axis_index("c")
    row0 = wid * (n_chunks * CHUNK)

    pltpu.sync_copy(idx_hbm.at[wid], idx_v)

    bufs = (buf0, buf1)
    sems = (sem0, sem1)

    def start(j, b):
        pltpu.async_copy(table_hbm.at[idx_v.at[j]], bufs[b], sems[b])

    for b in range(NBUF):
        start(b, b)

    def outer(g, carry):
        j0 = g * NBUF
        for b in range(NBUF):
            j = j0 + b
            pltpu.make_async_copy(table_hbm.at[idx_v.at[j]], bufs[b],
                                  sems[b]).wait()
            pltpu.sync_copy(bufs[b], out_hbm.at[pl.ds(row0 + j * CHUNK, CHUNK)])
            nxt = j + NBUF

            @pl.when(nxt < n_chunks)
            def _():
                start(nxt, b)
        return carry

    lax.fori_loop(0, n_chunks // NBUF, outer, 0)


@jax.jit
def _embed(idx, table):
    n_rows = idx.shape[0] * idx.shape[1] * idx.shape[2]
    mesh = plsc.VectorSubcoreMesh(core_axis_name="c", subcore_axis_name="s")
    f = pl.kernel(
        _embed_body,
        out_type=jax.ShapeDtypeStruct((n_rows, D_EMB), jnp.float32),
        mesh=mesh,
        scratch_types=[
            pltpu.VMEM(idx.shape[1:], jnp.int32),
            pltpu.VMEM((CHUNK, D_EMB), jnp.float32),
            pltpu.VMEM((CHUNK, D_EMB), jnp.float32),
            pltpu.SemaphoreType.DMA,
            pltpu.SemaphoreType.DMA,
        ],
        compiler_params=pltpu.CompilerParams(use_tc_tiling_on_sc=False),
    )
    return f(idx, table)


def kernel(txt_var, learned_embeddings):
    batch, hist = txt_var.shape
    n = batch * hist
    idx = txt_var.astype(jnp.int32).reshape(NW, n // (NW * CHUNK), CHUNK)
    out = _embed(idx, learned_embeddings)
    return out.reshape(batch, hist, D_EMB)


# speedup vs baseline: 1.0085x; 1.0085x over previous
import functools

import jax
import jax.numpy as jnp
from jax import lax
from jax.experimental import pallas as pl
from jax.experimental.pallas import tpu as pltpu
from jax.experimental.pallas import tpu_sc as plsc

D_EMB = 64
NC = 2
NS = 16
NW = NC * NS
CHUNK = 800
NBUF = 2


def _embed_body(idx_hbm, table_hbm, out_hbm, idx_v, buf0, buf1, sem0, sem1):
    n_chunks = idx_hbm.shape[1]
    wid = lax.axis_index("s") * NC + lax.axis_index("c")
    row0 = wid * (n_chunks * CHUNK)

    pltpu.sync_copy(idx_hbm.at[wid], idx_v)

    bufs = (buf0, buf1)
    sems = (sem0, sem1)

    def start(j, b):
        pltpu.async_copy(table_hbm.at[idx_v.at[j]], bufs[b], sems[b])

    for b in range(NBUF):
        start(b, b)

    def outer(g, carry):
        j0 = g * NBUF
        for b in range(NBUF):
            j = j0 + b
            pltpu.make_async_copy(table_hbm.at[idx_v.at[j]], bufs[b],
                                  sems[b]).wait()
            pltpu.sync_copy(bufs[b], out_hbm.at[pl.ds(row0 + j * CHUNK, CHUNK)])
            nxt = j + NBUF

            @pl.when(nxt < n_chunks)
            def _():
                start(nxt, b)
        return carry

    lax.fori_loop(0, n_chunks // NBUF, outer, 0)


@jax.jit
def _embed(idx, table):
    n_rows = idx.shape[0] * idx.shape[1] * idx.shape[2]
    mesh = plsc.VectorSubcoreMesh(core_axis_name="c", subcore_axis_name="s")
    f = pl.kernel(
        _embed_body,
        out_type=jax.ShapeDtypeStruct((n_rows, D_EMB), jnp.float32),
        mesh=mesh,
        scratch_types=[
            pltpu.VMEM(idx.shape[1:], jnp.int32),
            pltpu.VMEM((CHUNK, D_EMB), jnp.float32),
            pltpu.VMEM((CHUNK, D_EMB), jnp.float32),
            pltpu.SemaphoreType.DMA,
            pltpu.SemaphoreType.DMA,
        ],
        compiler_params=pltpu.CompilerParams(use_tc_tiling_on_sc=False),
    )
    return f(idx, table)


def kernel(txt_var, learned_embeddings):
    batch, hist = txt_var.shape
    n = batch * hist
    idx = txt_var.astype(jnp.int32).reshape(NW, n // (NW * CHUNK), CHUNK)
    out = _embed(idx, learned_embeddings)
    return out.reshape(batch, hist, D_EMB)
